# baseline (speedup 1.0000x reference)
import functools
import jax
import jax.numpy as jnp
from jax import lax
from jax.experimental import pallas as pl
from jax.experimental.pallas import tpu as pltpu
from jax.experimental.pallas import tpu_sc as plsc

_S3 = 3.0 ** 0.5
_EB = 512
_NW = 32
_CB = 128


def _sc_gather_rows(table, idx3):
    nw, ch, cb = idx3.shape
    rows_out = nw * ch * cb
    mesh = plsc.VectorSubcoreMesh(core_axis_name="c", subcore_axis_name="s")

    @functools.partial(
        pl.kernel,
        mesh=mesh,
        out_type=jax.ShapeDtypeStruct((rows_out, 128), jnp.float32),
        scratch_types=[
            pltpu.VMEM((ch, cb), jnp.int32),
            pltpu.VMEM((cb, 128), jnp.float32),
            pltpu.VMEM((cb, 128), jnp.float32),
            pltpu.SemaphoreType.DMA,
            pltpu.SemaphoreType.DMA,
        ],
    )
    def k(idx_hbm, table_hbm, out_hbm, idx_v, buf0, buf1, sem0, sem1):
        wid = lax.axis_index("s") * 2 + lax.axis_index("c")
        base = wid * (ch * cb)
        pltpu.sync_copy(idx_hbm.at[wid], idx_v)
        bufs = (buf0, buf1)
        sems = (sem0, sem1)
        cps = [None, None]
        cps[0] = pltpu.async_copy(table_hbm.at[idx_v.at[0]], bufs[0], sems[0])
        for c in range(ch):
            if c + 1 < ch:
                nxt = (c + 1) % 2
                cps[nxt] = pltpu.async_copy(
                    table_hbm.at[idx_v.at[c + 1]], bufs[nxt], sems[nxt])
            cps[c % 2].wait()
            pltpu.sync_copy(bufs[c % 2], out_hbm.at[pl.ds(base + c * cb, cb)])

    return k(idx3, table)


def _tc_body(ef_ref, src_ref, dst_ref, shT_ref, nodeT_ref, l1fT_ref,
             WpT_ref, bp_ref, Wqd0T_ref,
             lngk_ref, lnbk_ref, Wk1T_ref, bk1_ref, Wk2T_ref, bk2_ref,
             lngv_ref, lnbv_ref, Wv1T_ref, bv1_ref, Wv2T_ref, bv2_ref,
             WhoT_ref, WpnT_ref, bpnT_ref,
             Wqd1_ref, Whoo_ref,
             outT_ref, l1outT_ref,
             xq_s, acc_s, y_s, wk_s, wv_s, xs_s, qd_s, xqb_s):
    step = pl.program_id(0)
    nstep = pl.num_programs(0)
    f32 = jnp.float32

    @pl.when(step == 0)
    def _pre():
        x0T = jnp.dot(WpT_ref[...], nodeT_ref[...],
                      preferred_element_type=f32) + bp_ref[...]
        x1T = l1fT_ref[...]
        qd0T = jnp.dot(Wqd0T_ref[...], x0T, preferred_element_type=f32)
        xq_s[0:16, :] = x0T
        xq_s[16:28, :] = x1T
        xq_s[28:32, :] = jnp.zeros((4, 768), f32)
        xq_s[32:48, :] = qd0T
        for v in range(4):
            for i in range(3):
                r = sum(Wqd1_ref[w, v] * x1T[3 * w + i:3 * w + i + 1, :]
                        for w in range(4))
                xq_s[48 + 3 * v + i:49 + 3 * v + i, :] = r
        xq_s[60:64, :] = jnp.zeros((4, 768), f32)
        xqb_s[...] = xq_s[...].astype(jnp.bfloat16)
        acc_s[...] = jnp.zeros((48, 768), f32)
        y_s[28:32, :] = jnp.zeros((4, _EB), jnp.bfloat16)
        y_s[46:48, :] = jnp.zeros((2, _EB), jnp.bfloat16)

    efb = ef_ref[...]
    mu = jnp.mean(efb, axis=1, keepdims=True)
    var = jnp.mean((efb - mu) ** 2, axis=1, keepdims=True)
    nrm = (efb - mu) * lax.rsqrt(var + 1e-5)
    tdims = (((1,), (1,)), ((), ()))
    bf16 = jnp.bfloat16

    lnk = (nrm * lngk_ref[...] + lnbk_ref[...]).astype(bf16)
    hk = jnp.maximum(lax.dot_general(Wk1T_ref[...].astype(bf16), lnk, tdims,
                                     preferred_element_type=f32)
                     + bk1_ref[...], 0.0)
    wk_s[...] = jnp.dot(Wk2T_ref[...].astype(bf16), hk.astype(bf16),
                        preferred_element_type=f32) + bk2_ref[...]

    lnv = (nrm * lngv_ref[...] + lnbv_ref[...]).astype(bf16)
    hv = jnp.maximum(lax.dot_general(Wv1T_ref[...].astype(bf16), lnv, tdims,
                                     preferred_element_type=f32)
                     + bv1_ref[...], 0.0)
    wv_s[...] = jnp.dot(Wv2T_ref[...].astype(bf16), hv.astype(bf16),
                        preferred_element_type=f32) + bv2_ref[...]

    srcb = src_ref[0]
    dstb = dst_ref[0]
    iota = lax.broadcasted_iota(jnp.int32, (768, _EB), 0)
    ohs = (iota == srcb).astype(bf16)
    ohd = (iota == dstb).astype(bf16)
    xs_s[...] = jnp.dot(xqb_s[0:28, :], ohs, preferred_element_type=f32)
    qd_s[...] = jnp.dot(xqb_s[32:60, :], ohd, preferred_element_type=f32)
    xsT = xs_s
    qdT = qd_s
    wkT = wk_s
    wvT = wv_s

    shT = shT_ref[...]
    sh0 = shT[0:1]
    sh1 = [shT[1 + i:2 + i] for i in range(3)]
    a_ = shT[4:5]; b_ = shT[5:6]; c_ = shT[6:7]; d_ = shT[7:8]; e_ = shT[8:9]

    x0T = xsT[0:16]
    x1r = [xsT[16 + k:17 + k] for k in range(12)]
    x0sT = x0T * sh0
    inv3 = 1.0 / _S3
    dot11r = [(x1r[3*u] * sh1[0] + x1r[3*u+1] * sh1[1] + x1r[3*u+2] * sh1[2])
              * inv3 for u in range(4)]
    m00 = e_ - c_ * inv3
    m11 = -e_ - c_ * inv3
    m22 = c_ * (2.0 * inv3)
    i5 = 1.0 / (5.0 ** 0.5)
    i2 = 1.0 / (2.0 ** 0.5)
    m12r, y1r, crr = [], [], []
    for u in range(4):
        p0, p1, p2 = x1r[3*u], x1r[3*u+1], x1r[3*u+2]
        m12r += [(m00*p0 + a_*p1 + d_*p2) * i5,
                 (a_*p0 + m11*p1 + b_*p2) * i5,
                 (d_*p0 + b_*p1 + m22*p2) * i5]
        y1r += [p0 * sh0, p1 * sh0, p2 * sh0]
        crr += [(p1*sh1[2] - p2*sh1[1]) * i2,
                (p2*sh1[0] - p0*sh1[2]) * i2,
                (p0*sh1[1] - p1*sh1[0]) * i2]

    i20 = 1.0 / (20.0 ** 0.5)
    i24 = 1.0 / (24.0 ** 0.5)

    def fctp(wT, need_1e):
        out0 = sum(wT[16*u:16*u+16] * x0sT[u:u+1] for u in range(16))
        out0 = out0 + sum(wT[256+16*u:256+16*u+16] * dot11r[u]
                          for u in range(4))
        out0 = out0 * i20
        t011 = [sum(wT[320+4*u+v:321+4*u+v] * x0T[u:u+1] for u in range(16))
                for v in range(4)]
        o1o = []
        for v in range(4):
            for i in range(3):
                r = t011[v] * sh1[i]
                r = r + sum(wT[384+4*u+v:385+4*u+v] * y1r[3*u+i]
                            for u in range(4))
                r = r + sum(wT[400+4*u+v:401+4*u+v] * m12r[3*u+i]
                            for u in range(4))
                o1o.append(r * i24)
        if not need_1e:
            return out0, o1o, None
        o1e = [sum(wT[416+4*u+v:417+4*u+v] * crr[3*u+i] for u in range(4))
               * 0.5
               for v in range(4) for i in range(3)]
        return out0, o1o, o1e

    k0, k1o, _ = fctp(wkT, False)
    v0, v1o, v1e = fctp(wvT, True)

    d0 = jnp.sum(qdT[0:16] * k0, axis=0, keepdims=True)
    d1 = sum(qdT[16+k:17+k] * k1o[k] for k in range(12)) * inv3
    dd = (d0 + d1) * (1.0 / (288.0 ** 0.5))
    ex = jnp.exp(dd * (1.0 / (40.0 ** 0.5)))

    y_s[0:16, :] = (v0 * ex).astype(bf16)
    for k in range(12):
        y_s[16 + k:17 + k, :] = (v1o[k] * ex).astype(bf16)
        y_s[32 + k:33 + k, :] = (v1e[k] * ex).astype(bf16)
    y_s[44:45, :] = ex.astype(bf16)
    y_s[45:46, :] = jnp.ones((1, _EB), bf16)
    acc_s[...] = acc_s[...] + lax.dot_general(
        y_s[...], ohd, (((1,), (1,)), ((), ())), preferred_element_type=f32)

    @pl.when(step == nstep - 1)
    def _post():
        acc = acc_s[...]
        z = acc[44:45]
        cnt = acc[45:46]
        scale = 1.0 / ((z + 1e-5) * jnp.maximum(cnt, 1.0))
        o0T = acc[0:16] * scale
        f0T = jnp.dot(WhoT_ref[...], o0T, preferred_element_type=f32)
        node_outT = (jnp.dot(WpnT_ref[...], f0T, preferred_element_type=f32)
                     + bpnT_ref[...] + nodeT_ref[...])
        outT_ref[...] = node_outT
        for v in range(4):
            for i in range(3):
                r = sum(Whoo_ref[w, v] * acc[16+3*w+i:17+3*w+i]
                        for w in range(4)) * scale
                l1outT_ref[3*v+i:3*v+i+1, :] = r + l1fT_ref[3*v+i:3*v+i+1, :]


def _tc_main(ef, src3, dst3, shT, nodeT, l1fT, consts):
    (WpT, bp_c, Wqd0T, lngk, lnbk, Wk1T, bk1c, Wk2T, bk2c,
     lngv, lnbv, Wv1T, bv1c, Wv2T, bv2c, WhoT, WpnT, bpnT,
     Wqd1, Whoo) = consts
    nblk = ef.shape[0] // _EB
    whole = lambda s: pl.BlockSpec(s, lambda i: (0, 0))
    in_specs = [
        pl.BlockSpec((_EB, 128), lambda i: (i, 0)),
        pl.BlockSpec((1, 1, _EB), lambda i: (i, 0, 0)),
        pl.BlockSpec((1, 1, _EB), lambda i: (i, 0, 0)),
        pl.BlockSpec((9, _EB), lambda i: (0, i)),
        whole((256, 768)), whole((12, 768)),
        whole(WpT.shape), whole(bp_c.shape), whole(Wqd0T.shape),
        whole(lngk.shape), whole(lnbk.shape), whole(Wk1T.shape),
        whole(bk1c.shape), whole(Wk2T.shape), whole(bk2c.shape),
        whole(lngv.shape), whole(lnbv.shape), whole(Wv1T.shape),
        whole(bv1c.shape), whole(Wv2T.shape), whole(bv2c.shape),
        whole(WhoT.shape), whole(WpnT.shape), whole(bpnT.shape),
        pl.BlockSpec(memory_space=pltpu.SMEM),
        pl.BlockSpec(memory_space=pltpu.SMEM),
    ]
    out_specs = [whole((256, 768)), whole((12, 768))]
    return pl.pallas_call(
        _tc_body,
        grid=(nblk,),
        in_specs=in_specs,
        out_specs=out_specs,
        out_shape=[jax.ShapeDtypeStruct((256, 768), jnp.float32),
                   jax.ShapeDtypeStruct((12, 768), jnp.float32)],
        scratch_shapes=[pltpu.VMEM((64, 768), jnp.float32),
                        pltpu.VMEM((48, 768), jnp.float32),
                        pltpu.VMEM((48, _EB), jnp.bfloat16),
                        pltpu.VMEM((432, _EB), jnp.float32),
                        pltpu.VMEM((432, _EB), jnp.float32),
                        pltpu.VMEM((28, _EB), jnp.float32),
                        pltpu.VMEM((28, _EB), jnp.float32),
                        pltpu.VMEM((64, 768), jnp.bfloat16)],
        compiler_params=pltpu.CompilerParams(
            dimension_semantics=("arbitrary",)),
    )(ef, src3, dst3, shT, nodeT, l1fT,
      WpT, bp_c, Wqd0T, lngk, lnbk, Wk1T, bk1c, Wk2T, bk2c,
      lngv, lnbv, Wv1T, bv1c, Wv2T, bv2c, WhoT, WpnT, bpnT, Wqd1, Whoo)


def kernel(node, pair, l1_feats, pair_index, edge_src, edge_dst, edge_sh,
           Wp, bp, Wq0, Wq1, ln_g_k, ln_b_k, Wk1, bk1, Wk2, bk2,
           ln_g_v, ln_b_v, Wv1, bv1, Wv2, bv2, Wd0, Wd1o, Wd1e,
           Wh0, Wh1o, Wh1e, Wo0, Wo1o, Wpn, bpn):
    B, L = node.shape[0], node.shape[1]
    n = B * L
    E = edge_src.shape[0]

    flat = (pair_index[0] * (L * L) + pair_index[1] * L
            + pair_index[2]).astype(jnp.int32)
    ch = E // (_NW * _CB)
    ef = _sc_gather_rows(pair.reshape(B * L * L, 128),
                         flat.reshape(_NW, ch, _CB))

    f32 = jnp.float32
    src3 = edge_src.reshape(E // _EB, 1, _EB)
    dst3 = edge_dst.reshape(E // _EB, 1, _EB)
    shT = edge_sh.T
    nodeT = node.reshape(n, 256).T
    l1fT = l1_feats.reshape(n, 12).T
    consts = (
        Wp.T, bp.reshape(16, 1),
        (Wq0 @ Wd0).T / 4.0,
        ln_g_k.reshape(1, 128), ln_b_k.reshape(1, 128),
        Wk1.T, bk1.reshape(128, 1), Wk2.T, bk2.reshape(432, 1),
        ln_g_v.reshape(1, 128), ln_b_v.reshape(1, 128),
        Wv1.T, bv1.reshape(128, 1), Wv2.T, bv2.reshape(432, 1),
        (Wh0 @ Wo0).T / 16.0,
        Wpn.T, bpn.reshape(256, 1),
        (Wq1 @ Wd1o) / 2.0,
        (Wh1o @ Wo1o) / 4.0,
    )
    consts = tuple(c.astype(f32) for c in consts)

    outT, l1outT = _tc_main(ef, src3, dst3, shT, nodeT, l1fT, consts)
    node_out = outT.T.reshape(B, L, 256)
    l1_out = l1outT.T.reshape(B, L, 12)
    return node_out, l1_out


# speedup vs baseline: 1.1462x; 1.1462x over previous
import functools
import jax
import jax.numpy as jnp
from jax import lax
from jax.experimental import pallas as pl
from jax.experimental.pallas import tpu as pltpu
from jax.experimental.pallas import tpu_sc as plsc

_S3 = 3.0 ** 0.5
_EB = 1024
_NW = 32
_CB = 128


def _sc_gather_rows(table, idx3):
    nw, ch, cb = idx3.shape
    rows_out = nw * ch * cb
    mesh = plsc.VectorSubcoreMesh(core_axis_name="c", subcore_axis_name="s")

    @functools.partial(
        pl.kernel,
        mesh=mesh,
        out_type=jax.ShapeDtypeStruct((rows_out, 128), jnp.float32),
        scratch_types=[
            pltpu.VMEM((ch, cb), jnp.int32),
            pltpu.VMEM((cb, 128), jnp.float32),
            pltpu.VMEM((cb, 128), jnp.float32),
            pltpu.SemaphoreType.DMA,
            pltpu.SemaphoreType.DMA,
        ],
    )
    def k(idx_hbm, table_hbm, out_hbm, idx_v, buf0, buf1, sem0, sem1):
        wid = lax.axis_index("s") * 2 + lax.axis_index("c")
        base = wid * (ch * cb)
        pltpu.sync_copy(idx_hbm.at[wid], idx_v)
        bufs = (buf0, buf1)
        sems = (sem0, sem1)
        cps = [None, None]
        cps[0] = pltpu.async_copy(table_hbm.at[idx_v.at[0]], bufs[0], sems[0])
        for c in range(ch):
            if c + 1 < ch:
                nxt = (c + 1) % 2
                cps[nxt] = pltpu.async_copy(
                    table_hbm.at[idx_v.at[c + 1]], bufs[nxt], sems[nxt])
            cps[c % 2].wait()
            pltpu.sync_copy(bufs[c % 2], out_hbm.at[pl.ds(base + c * cb, cb)])

    return k(idx3, table)


def _tc_body(ef_ref, src_ref, dst_ref, shT_ref, nodeT_ref, l1fT_ref,
             WpT_ref, bp_ref, Wqd0T_ref,
             lngk_ref, lnbk_ref, Wk1T_ref, bk1_ref, Wk2T_ref, bk2_ref,
             lngv_ref, lnbv_ref, Wv1T_ref, bv1_ref, Wv2T_ref, bv2_ref,
             WhoT_ref, WpnT_ref, bpnT_ref,
             Wqd1_ref, Whoo_ref,
             outT_ref, l1outT_ref,
             xq_s, acc_s, y_s, wk_s, wv_s, xs_s, qd_s):
    step = pl.program_id(0)
    nstep = pl.num_programs(0)
    f32 = jnp.float32

    @pl.when(step == 0)
    def _pre():
        x0T = jnp.dot(WpT_ref[...], nodeT_ref[...],
                      preferred_element_type=f32) + bp_ref[...]
        x1T = l1fT_ref[...]
        qd0T = jnp.dot(Wqd0T_ref[...], x0T, preferred_element_type=f32)
        xq_s[0:16, :] = x0T
        xq_s[16:28, :] = x1T
        xq_s[28:32, :] = jnp.zeros((4, 768), f32)
        xq_s[32:48, :] = qd0T
        for v in range(4):
            for i in range(3):
                r = sum(Wqd1_ref[w, v] * x1T[3 * w + i:3 * w + i + 1, :]
                        for w in range(4))
                xq_s[48 + 3 * v + i:49 + 3 * v + i, :] = r
        xq_s[60:64, :] = jnp.zeros((4, 768), f32)
        acc_s[...] = jnp.zeros((48, 768), f32)
        y_s[28:32, :] = jnp.zeros((4, _EB), f32)
        y_s[46:48, :] = jnp.zeros((2, _EB), f32)

    efb = ef_ref[...]
    mu = jnp.mean(efb, axis=1, keepdims=True)
    var = jnp.mean((efb - mu) ** 2, axis=1, keepdims=True)
    nrm = (efb - mu) * lax.rsqrt(var + 1e-5)
    tdims = (((1,), (1,)), ((), ()))
    bf16 = jnp.bfloat16

    lnk = (nrm * lngk_ref[...] + lnbk_ref[...]).astype(bf16)
    hk = jnp.maximum(lax.dot_general(Wk1T_ref[...].astype(bf16), lnk, tdims,
                                     preferred_element_type=f32)
                     + bk1_ref[...], 0.0)
    wk_s[...] = jnp.dot(Wk2T_ref[...].astype(bf16), hk.astype(bf16),
                        preferred_element_type=f32) + bk2_ref[...]

    lnv = (nrm * lngv_ref[...] + lnbv_ref[...]).astype(bf16)
    hv = jnp.maximum(lax.dot_general(Wv1T_ref[...].astype(bf16), lnv, tdims,
                                     preferred_element_type=f32)
                     + bv1_ref[...], 0.0)
    wv_s[...] = jnp.dot(Wv2T_ref[...].astype(bf16), hv.astype(bf16),
                        preferred_element_type=f32) + bv2_ref[...]

    srcb = src_ref[0]
    dstb = dst_ref[0]
    iota = lax.broadcasted_iota(jnp.int32, (768, _EB), 0)
    ohs = (iota == srcb).astype(f32)
    ohd = (iota == dstb).astype(f32)
    xs_s[...] = jnp.dot(xq_s[0:28, :], ohs, preferred_element_type=f32)
    qd_s[...] = jnp.dot(xq_s[32:60, :], ohd, preferred_element_type=f32)
    xsT = xs_s
    qdT = qd_s
    wkT = wk_s
    wvT = wv_s

    shT = shT_ref[...]
    sh0 = shT[0:1]
    sh1 = [shT[1 + i:2 + i] for i in range(3)]
    a_ = shT[4:5]; b_ = shT[5:6]; c_ = shT[6:7]; d_ = shT[7:8]; e_ = shT[8:9]

    x0T = xsT[0:16]
    x1r = [xsT[16 + k:17 + k] for k in range(12)]
    x0sT = x0T * sh0
    inv3 = 1.0 / _S3
    dot11r = [(x1r[3*u] * sh1[0] + x1r[3*u+1] * sh1[1] + x1r[3*u+2] * sh1[2])
              * inv3 for u in range(4)]
    m00 = e_ - c_ * inv3
    m11 = -e_ - c_ * inv3
    m22 = c_ * (2.0 * inv3)
    i5 = 1.0 / (5.0 ** 0.5)
    i2 = 1.0 / (2.0 ** 0.5)
    m12r, y1r, crr = [], [], []
    for u in range(4):
        p0, p1, p2 = x1r[3*u], x1r[3*u+1], x1r[3*u+2]
        m12r += [(m00*p0 + a_*p1 + d_*p2) * i5,
                 (a_*p0 + m11*p1 + b_*p2) * i5,
                 (d_*p0 + b_*p1 + m22*p2) * i5]
        y1r += [p0 * sh0, p1 * sh0, p2 * sh0]
        crr += [(p1*sh1[2] - p2*sh1[1]) * i2,
                (p2*sh1[0] - p0*sh1[2]) * i2,
                (p0*sh1[1] - p1*sh1[0]) * i2]

    i20 = 1.0 / (20.0 ** 0.5)
    i24 = 1.0 / (24.0 ** 0.5)

    def fctp(wT, need_1e):
        out0 = sum(wT[16*u:16*u+16] * x0sT[u:u+1] for u in range(16))
        out0 = out0 + sum(wT[256+16*u:256+16*u+16] * dot11r[u]
                          for u in range(4))
        out0 = out0 * i20
        t011 = [sum(wT[320+4*u+v:321+4*u+v] * x0T[u:u+1] for u in range(16))
                for v in range(4)]
        o1o = []
        for v in range(4):
            for i in range(3):
                r = t011[v] * sh1[i]
                r = r + sum(wT[384+4*u+v:385+4*u+v] * y1r[3*u+i]
                            for u in range(4))
                r = r + sum(wT[400+4*u+v:401+4*u+v] * m12r[3*u+i]
                            for u in range(4))
                o1o.append(r * i24)
        if not need_1e:
            return out0, o1o, None
        o1e = [sum(wT[416+4*u+v:417+4*u+v] * crr[3*u+i] for u in range(4))
               * 0.5
               for v in range(4) for i in range(3)]
        return out0, o1o, o1e

    k0, k1o, _ = fctp(wkT, False)
    v0, v1o, v1e = fctp(wvT, True)

    d0 = jnp.sum(qdT[0:16] * k0, axis=0, keepdims=True)
    d1 = sum(qdT[16+k:17+k] * k1o[k] for k in range(12)) * inv3
    dd = (d0 + d1) * (1.0 / (288.0 ** 0.5))
    ex = jnp.exp(dd * (1.0 / (40.0 ** 0.5)))

    y_s[0:16, :] = v0 * ex
    for k in range(12):
        y_s[16 + k:17 + k, :] = v1o[k] * ex
        y_s[32 + k:33 + k, :] = v1e[k] * ex
    y_s[44:45, :] = ex
    y_s[45:46, :] = jnp.ones((1, _EB), f32)
    acc_s[...] = acc_s[...] + lax.dot_general(
        y_s[...], ohd, (((1,), (1,)), ((), ())), preferred_element_type=f32)

    @pl.when(step == nstep - 1)
    def _post():
        acc = acc_s[...]
        z = acc[44:45]
        cnt = acc[45:46]
        scale = 1.0 / ((z + 1e-5) * jnp.maximum(cnt, 1.0))
        o0T = acc[0:16] * scale
        f0T = jnp.dot(WhoT_ref[...], o0T, preferred_element_type=f32)
        node_outT = (jnp.dot(WpnT_ref[...], f0T, preferred_element_type=f32)
                     + bpnT_ref[...] + nodeT_ref[...])
        outT_ref[...] = node_outT
        for v in range(4):
            for i in range(3):
                r = sum(Whoo_ref[w, v] * acc[16+3*w+i:17+3*w+i]
                        for w in range(4)) * scale
                l1outT_ref[3*v+i:3*v+i+1, :] = r + l1fT_ref[3*v+i:3*v+i+1, :]


def _tc_main(ef, src3, dst3, shT, nodeT, l1fT, consts):
    (WpT, bp_c, Wqd0T, lngk, lnbk, Wk1T, bk1c, Wk2T, bk2c,
     lngv, lnbv, Wv1T, bv1c, Wv2T, bv2c, WhoT, WpnT, bpnT,
     Wqd1, Whoo) = consts
    nblk = ef.shape[0] // _EB
    whole = lambda s: pl.BlockSpec(s, lambda i: (0, 0))
    in_specs = [
        pl.BlockSpec((_EB, 128), lambda i: (i, 0)),
        pl.BlockSpec((1, 1, _EB), lambda i: (i, 0, 0)),
        pl.BlockSpec((1, 1, _EB), lambda i: (i, 0, 0)),
        pl.BlockSpec((9, _EB), lambda i: (0, i)),
        whole((256, 768)), whole((12, 768)),
        whole(WpT.shape), whole(bp_c.shape), whole(Wqd0T.shape),
        whole(lngk.shape), whole(lnbk.shape), whole(Wk1T.shape),
        whole(bk1c.shape), whole(Wk2T.shape), whole(bk2c.shape),
        whole(lngv.shape), whole(lnbv.shape), whole(Wv1T.shape),
        whole(bv1c.shape), whole(Wv2T.shape), whole(bv2c.shape),
        whole(WhoT.shape), whole(WpnT.shape), whole(bpnT.shape),
        pl.BlockSpec(memory_space=pltpu.SMEM),
        pl.BlockSpec(memory_space=pltpu.SMEM),
    ]
    out_specs = [whole((256, 768)), whole((12, 768))]
    return pl.pallas_call(
        _tc_body,
        grid=(nblk,),
        in_specs=in_specs,
        out_specs=out_specs,
        out_shape=[jax.ShapeDtypeStruct((256, 768), jnp.float32),
                   jax.ShapeDtypeStruct((12, 768), jnp.float32)],
        scratch_shapes=[pltpu.VMEM((64, 768), jnp.float32),
                        pltpu.VMEM((48, 768), jnp.float32),
                        pltpu.VMEM((48, _EB), jnp.float32),
                        pltpu.VMEM((432, _EB), jnp.float32),
                        pltpu.VMEM((432, _EB), jnp.float32),
                        pltpu.VMEM((28, _EB), jnp.float32),
                        pltpu.VMEM((28, _EB), jnp.float32)],
        compiler_params=pltpu.CompilerParams(
            dimension_semantics=("arbitrary",)),
    )(ef, src3, dst3, shT, nodeT, l1fT,
      WpT, bp_c, Wqd0T, lngk, lnbk, Wk1T, bk1c, Wk2T, bk2c,
      lngv, lnbv, Wv1T, bv1c, Wv2T, bv2c, WhoT, WpnT, bpnT, Wqd1, Whoo)


def kernel(node, pair, l1_feats, pair_index, edge_src, edge_dst, edge_sh,
           Wp, bp, Wq0, Wq1, ln_g_k, ln_b_k, Wk1, bk1, Wk2, bk2,
           ln_g_v, ln_b_v, Wv1, bv1, Wv2, bv2, Wd0, Wd1o, Wd1e,
           Wh0, Wh1o, Wh1e, Wo0, Wo1o, Wpn, bpn):
    B, L = node.shape[0], node.shape[1]
    n = B * L
    E = edge_src.shape[0]

    flat = (pair_index[0] * (L * L) + pair_index[1] * L
            + pair_index[2]).astype(jnp.int32)
    ch = E // (_NW * _CB)
    ef = _sc_gather_rows(pair.reshape(B * L * L, 128),
                         flat.reshape(_NW, ch, _CB))

    f32 = jnp.float32
    src3 = edge_src.reshape(E // _EB, 1, _EB)
    dst3 = edge_dst.reshape(E // _EB, 1, _EB)
    shT = edge_sh.T
    nodeT = node.reshape(n, 256).T
    l1fT = l1_feats.reshape(n, 12).T
    consts = (
        Wp.T, bp.reshape(16, 1),
        (Wq0 @ Wd0).T / 4.0,
        ln_g_k.reshape(1, 128), ln_b_k.reshape(1, 128),
        Wk1.T, bk1.reshape(128, 1), Wk2.T, bk2.reshape(432, 1),
        ln_g_v.reshape(1, 128), ln_b_v.reshape(1, 128),
        Wv1.T, bv1.reshape(128, 1), Wv2.T, bv2.reshape(432, 1),
        (Wh0 @ Wo0).T / 16.0,
        Wpn.T, bpn.reshape(256, 1),
        (Wq1 @ Wd1o) / 2.0,
        (Wh1o @ Wo1o) / 4.0,
    )
    consts = tuple(c.astype(f32) for c in consts)

    outT, l1outT = _tc_main(ef, src3, dst3, shT, nodeT, l1fT, consts)
    node_out = outT.T.reshape(B, L, 256)
    l1_out = l1outT.T.reshape(B, L, 12)
    return node_out, l1_out


# speedup vs baseline: 1.2988x; 1.1332x over previous
import functools
import jax
import jax.numpy as jnp
from jax import lax
from jax.experimental import pallas as pl
from jax.experimental.pallas import tpu as pltpu
from jax.experimental.pallas import tpu_sc as plsc

_S3 = 3.0 ** 0.5
_EB = 2048
_NW = 32
_CB = 128


def _sc_gather_rows(table, idx3):
    nw, ch, cb = idx3.shape
    rows_out = nw * ch * cb
    mesh = plsc.VectorSubcoreMesh(core_axis_name="c", subcore_axis_name="s")

    @functools.partial(
        pl.kernel,
        mesh=mesh,
        out_type=jax.ShapeDtypeStruct((rows_out, 128), jnp.float32),
        scratch_types=[
            pltpu.VMEM((ch, cb), jnp.int32),
            pltpu.VMEM((cb, 128), jnp.float32),
            pltpu.VMEM((cb, 128), jnp.float32),
            pltpu.SemaphoreType.DMA,
            pltpu.SemaphoreType.DMA,
        ],
    )
    def k(idx_hbm, table_hbm, out_hbm, idx_v, buf0, buf1, sem0, sem1):
        wid = lax.axis_index("s") * 2 + lax.axis_index("c")
        base = wid * (ch * cb)
        pltpu.sync_copy(idx_hbm.at[wid], idx_v)
        bufs = (buf0, buf1)
        sems = (sem0, sem1)
        cps = [None, None]
        cps[0] = pltpu.async_copy(table_hbm.at[idx_v.at[0]], bufs[0], sems[0])
        for c in range(ch):
            if c + 1 < ch:
                nxt = (c + 1) % 2
                cps[nxt] = pltpu.async_copy(
                    table_hbm.at[idx_v.at[c + 1]], bufs[nxt], sems[nxt])
            cps[c % 2].wait()
            pltpu.sync_copy(bufs[c % 2], out_hbm.at[pl.ds(base + c * cb, cb)])

    return k(idx3, table)


def _tc_body(ef_ref, src_ref, dst_ref, shT_ref, nodeT_ref, l1fT_ref,
             WpT_ref, bp_ref, Wqd0T_ref,
             lngk_ref, lnbk_ref, Wk1T_ref, bk1_ref, Wk2T_ref, bk2_ref,
             lngv_ref, lnbv_ref, Wv1T_ref, bv1_ref, Wv2T_ref, bv2_ref,
             WhoT_ref, WpnT_ref, bpnT_ref,
             Wqd1_ref, Whoo_ref,
             outT_ref, l1outT_ref,
             xq_s, acc_s, y_s, wk_s, wv_s, xs_s, qd_s):
    step = pl.program_id(0)
    nstep = pl.num_programs(0)
    f32 = jnp.float32

    @pl.when(step == 0)
    def _pre():
        x0T = jnp.dot(WpT_ref[...], nodeT_ref[...],
                      preferred_element_type=f32) + bp_ref[...]
        x1T = l1fT_ref[...]
        qd0T = jnp.dot(Wqd0T_ref[...], x0T, preferred_element_type=f32)
        xq_s[0:16, :] = x0T
        xq_s[16:28, :] = x1T
        xq_s[28:32, :] = jnp.zeros((4, 768), f32)
        xq_s[32:48, :] = qd0T
        for v in range(4):
            for i in range(3):
                r = sum(Wqd1_ref[w, v] * x1T[3 * w + i:3 * w + i + 1, :]
                        for w in range(4))
                xq_s[48 + 3 * v + i:49 + 3 * v + i, :] = r
        xq_s[60:64, :] = jnp.zeros((4, 768), f32)
        acc_s[...] = jnp.zeros((48, 768), f32)
        y_s[28:32, :] = jnp.zeros((4, _EB), f32)
        y_s[46:48, :] = jnp.zeros((2, _EB), f32)

    efb = ef_ref[...]
    mu = jnp.mean(efb, axis=1, keepdims=True)
    var = jnp.mean((efb - mu) ** 2, axis=1, keepdims=True)
    nrm = (efb - mu) * lax.rsqrt(var + 1e-5)
    tdims = (((1,), (1,)), ((), ()))
    bf16 = jnp.bfloat16

    lnk = (nrm * lngk_ref[...] + lnbk_ref[...]).astype(bf16)
    hk = jnp.maximum(lax.dot_general(Wk1T_ref[...].astype(bf16), lnk, tdims,
                                     preferred_element_type=f32)
                     + bk1_ref[...], 0.0)
    wk_s[...] = jnp.dot(Wk2T_ref[...].astype(bf16), hk.astype(bf16),
                        preferred_element_type=f32) + bk2_ref[...]

    lnv = (nrm * lngv_ref[...] + lnbv_ref[...]).astype(bf16)
    hv = jnp.maximum(lax.dot_general(Wv1T_ref[...].astype(bf16), lnv, tdims,
                                     preferred_element_type=f32)
                     + bv1_ref[...], 0.0)
    wv_s[...] = jnp.dot(Wv2T_ref[...].astype(bf16), hv.astype(bf16),
                        preferred_element_type=f32) + bv2_ref[...]

    srcb = src_ref[0]
    dstb = dst_ref[0]
    iota = lax.broadcasted_iota(jnp.int32, (768, _EB), 0)
    ohs = (iota == srcb).astype(f32)
    ohd = (iota == dstb).astype(f32)
    xs_s[...] = jnp.dot(xq_s[0:28, :], ohs, preferred_element_type=f32)
    qd_s[...] = jnp.dot(xq_s[32:60, :], ohd, preferred_element_type=f32)
    xsT = xs_s
    qdT = qd_s
    wkT = wk_s
    wvT = wv_s

    shT = shT_ref[...]
    sh0 = shT[0:1]
    sh1 = [shT[1 + i:2 + i] for i in range(3)]
    a_ = shT[4:5]; b_ = shT[5:6]; c_ = shT[6:7]; d_ = shT[7:8]; e_ = shT[8:9]

    x0T = xsT[0:16]
    x1r = [xsT[16 + k:17 + k] for k in range(12)]
    x0sT = x0T * sh0
    inv3 = 1.0 / _S3
    dot11r = [(x1r[3*u] * sh1[0] + x1r[3*u+1] * sh1[1] + x1r[3*u+2] * sh1[2])
              * inv3 for u in range(4)]
    m00 = e_ - c_ * inv3
    m11 = -e_ - c_ * inv3
    m22 = c_ * (2.0 * inv3)
    i5 = 1.0 / (5.0 ** 0.5)
    i2 = 1.0 / (2.0 ** 0.5)
    m12r, y1r, crr = [], [], []
    for u in range(4):
        p0, p1, p2 = x1r[3*u], x1r[3*u+1], x1r[3*u+2]
        m12r += [(m00*p0 + a_*p1 + d_*p2) * i5,
                 (a_*p0 + m11*p1 + b_*p2) * i5,
                 (d_*p0 + b_*p1 + m22*p2) * i5]
        y1r += [p0 * sh0, p1 * sh0, p2 * sh0]
        crr += [(p1*sh1[2] - p2*sh1[1]) * i2,
                (p2*sh1[0] - p0*sh1[2]) * i2,
                (p0*sh1[1] - p1*sh1[0]) * i2]

    i20 = 1.0 / (20.0 ** 0.5)
    i24 = 1.0 / (24.0 ** 0.5)

    def fctp(wT, need_1e):
        out0 = sum(wT[16*u:16*u+16] * x0sT[u:u+1] for u in range(16))
        out0 = out0 + sum(wT[256+16*u:256+16*u+16] * dot11r[u]
                          for u in range(4))
        out0 = out0 * i20
        t011 = [sum(wT[320+4*u+v:321+4*u+v] * x0T[u:u+1] for u in range(16))
                for v in range(4)]
        o1o = []
        for v in range(4):
            for i in range(3):
                r = t011[v] * sh1[i]
                r = r + sum(wT[384+4*u+v:385+4*u+v] * y1r[3*u+i]
                            for u in range(4))
                r = r + sum(wT[400+4*u+v:401+4*u+v] * m12r[3*u+i]
                            for u in range(4))
                o1o.append(r * i24)
        if not need_1e:
            return out0, o1o, None
        o1e = [sum(wT[416+4*u+v:417+4*u+v] * crr[3*u+i] for u in range(4))
               * 0.5
               for v in range(4) for i in range(3)]
        return out0, o1o, o1e

    k0, k1o, _ = fctp(wkT, False)
    v0, v1o, v1e = fctp(wvT, True)

    d0 = jnp.sum(qdT[0:16] * k0, axis=0, keepdims=True)
    d1 = sum(qdT[16+k:17+k] * k1o[k] for k in range(12)) * inv3
    dd = (d0 + d1) * (1.0 / (288.0 ** 0.5))
    ex = jnp.exp(dd * (1.0 / (40.0 ** 0.5)))

    y_s[0:16, :] = v0 * ex
    for k in range(12):
        y_s[16 + k:17 + k, :] = v1o[k] * ex
        y_s[32 + k:33 + k, :] = v1e[k] * ex
    y_s[44:45, :] = ex
    y_s[45:46, :] = jnp.ones((1, _EB), f32)
    acc_s[...] = acc_s[...] + lax.dot_general(
        y_s[...], ohd, (((1,), (1,)), ((), ())), preferred_element_type=f32)

    @pl.when(step == nstep - 1)
    def _post():
        acc = acc_s[...]
        z = acc[44:45]
        cnt = acc[45:46]
        scale = 1.0 / ((z + 1e-5) * jnp.maximum(cnt, 1.0))
        o0T = acc[0:16] * scale
        f0T = jnp.dot(WhoT_ref[...], o0T, preferred_element_type=f32)
        node_outT = (jnp.dot(WpnT_ref[...], f0T, preferred_element_type=f32)
                     + bpnT_ref[...] + nodeT_ref[...])
        outT_ref[...] = node_outT
        for v in range(4):
            for i in range(3):
                r = sum(Whoo_ref[w, v] * acc[16+3*w+i:17+3*w+i]
                        for w in range(4)) * scale
                l1outT_ref[3*v+i:3*v+i+1, :] = r + l1fT_ref[3*v+i:3*v+i+1, :]


def _tc_main(ef, src3, dst3, shT, nodeT, l1fT, consts):
    (WpT, bp_c, Wqd0T, lngk, lnbk, Wk1T, bk1c, Wk2T, bk2c,
     lngv, lnbv, Wv1T, bv1c, Wv2T, bv2c, WhoT, WpnT, bpnT,
     Wqd1, Whoo) = consts
    nblk = ef.shape[0] // _EB
    whole = lambda s: pl.BlockSpec(s, lambda i: (0, 0))
    in_specs = [
        pl.BlockSpec((_EB, 128), lambda i: (i, 0)),
        pl.BlockSpec((1, 1, _EB), lambda i: (i, 0, 0)),
        pl.BlockSpec((1, 1, _EB), lambda i: (i, 0, 0)),
        pl.BlockSpec((9, _EB), lambda i: (0, i)),
        whole((256, 768)), whole((12, 768)),
        whole(WpT.shape), whole(bp_c.shape), whole(Wqd0T.shape),
        whole(lngk.shape), whole(lnbk.shape), whole(Wk1T.shape),
        whole(bk1c.shape), whole(Wk2T.shape), whole(bk2c.shape),
        whole(lngv.shape), whole(lnbv.shape), whole(Wv1T.shape),
        whole(bv1c.shape), whole(Wv2T.shape), whole(bv2c.shape),
        whole(WhoT.shape), whole(WpnT.shape), whole(bpnT.shape),
        pl.BlockSpec(memory_space=pltpu.SMEM),
        pl.BlockSpec(memory_space=pltpu.SMEM),
    ]
    out_specs = [whole((256, 768)), whole((12, 768))]
    return pl.pallas_call(
        _tc_body,
        grid=(nblk,),
        in_specs=in_specs,
        out_specs=out_specs,
        out_shape=[jax.ShapeDtypeStruct((256, 768), jnp.float32),
                   jax.ShapeDtypeStruct((12, 768), jnp.float32)],
        scratch_shapes=[pltpu.VMEM((64, 768), jnp.float32),
                        pltpu.VMEM((48, 768), jnp.float32),
                        pltpu.VMEM((48, _EB), jnp.float32),
                        pltpu.VMEM((432, _EB), jnp.float32),
                        pltpu.VMEM((432, _EB), jnp.float32),
                        pltpu.VMEM((28, _EB), jnp.float32),
                        pltpu.VMEM((28, _EB), jnp.float32)],
        compiler_params=pltpu.CompilerParams(
            dimension_semantics=("arbitrary",)),
    )(ef, src3, dst3, shT, nodeT, l1fT,
      WpT, bp_c, Wqd0T, lngk, lnbk, Wk1T, bk1c, Wk2T, bk2c,
      lngv, lnbv, Wv1T, bv1c, Wv2T, bv2c, WhoT, WpnT, bpnT, Wqd1, Whoo)


def kernel(node, pair, l1_feats, pair_index, edge_src, edge_dst, edge_sh,
           Wp, bp, Wq0, Wq1, ln_g_k, ln_b_k, Wk1, bk1, Wk2, bk2,
           ln_g_v, ln_b_v, Wv1, bv1, Wv2, bv2, Wd0, Wd1o, Wd1e,
           Wh0, Wh1o, Wh1e, Wo0, Wo1o, Wpn, bpn):
    B, L = node.shape[0], node.shape[1]
    n = B * L
    E = edge_src.shape[0]

    flat = (pair_index[0] * (L * L) + pair_index[1] * L
            + pair_index[2]).astype(jnp.int32)
    ch = E // (_NW * _CB)
    ef = _sc_gather_rows(pair.reshape(B * L * L, 128),
                         flat.reshape(_NW, ch, _CB))

    f32 = jnp.float32
    src3 = edge_src.reshape(E // _EB, 1, _EB)
    dst3 = edge_dst.reshape(E // _EB, 1, _EB)
    shT = edge_sh.T
    nodeT = node.reshape(n, 256).T
    l1fT = l1_feats.reshape(n, 12).T
    consts = (
        Wp.T, bp.reshape(16, 1),
        (Wq0 @ Wd0).T / 4.0,
        ln_g_k.reshape(1, 128), ln_b_k.reshape(1, 128),
        Wk1.T, bk1.reshape(128, 1), Wk2.T, bk2.reshape(432, 1),
        ln_g_v.reshape(1, 128), ln_b_v.reshape(1, 128),
        Wv1.T, bv1.reshape(128, 1), Wv2.T, bv2.reshape(432, 1),
        (Wh0 @ Wo0).T / 16.0,
        Wpn.T, bpn.reshape(256, 1),
        (Wq1 @ Wd1o) / 2.0,
        (Wh1o @ Wo1o) / 4.0,
    )
    consts = tuple(c.astype(f32) for c in consts)

    outT, l1outT = _tc_main(ef, src3, dst3, shT, nodeT, l1fT, consts)
    node_out = outT.T.reshape(B, L, 256)
    l1_out = l1outT.T.reshape(B, L, 12)
    return node_out, l1_out


# speedup vs baseline: 1.3726x; 1.0568x over previous
import functools
import jax
import jax.numpy as jnp
from jax import lax
from jax.experimental import pallas as pl
from jax.experimental.pallas import tpu as pltpu
from jax.experimental.pallas import tpu_sc as plsc

_S3 = 3.0 ** 0.5
_EB = 4096
_NW = 32
_CB = 128


def _sc_gather_rows(table, idx3):
    nw, ch, cb = idx3.shape
    rows_out = nw * ch * cb
    mesh = plsc.VectorSubcoreMesh(core_axis_name="c", subcore_axis_name="s")

    @functools.partial(
        pl.kernel,
        mesh=mesh,
        out_type=jax.ShapeDtypeStruct((rows_out, 128), jnp.float32),
        scratch_types=[
            pltpu.VMEM((ch, cb), jnp.int32),
            pltpu.VMEM((cb, 128), jnp.float32),
            pltpu.VMEM((cb, 128), jnp.float32),
            pltpu.SemaphoreType.DMA,
            pltpu.SemaphoreType.DMA,
        ],
    )
    def k(idx_hbm, table_hbm, out_hbm, idx_v, buf0, buf1, sem0, sem1):
        wid = lax.axis_index("s") * 2 + lax.axis_index("c")
        base = wid * (ch * cb)
        pltpu.sync_copy(idx_hbm.at[wid], idx_v)
        bufs = (buf0, buf1)
        sems = (sem0, sem1)
        cps = [None, None]
        cps[0] = pltpu.async_copy(table_hbm.at[idx_v.at[0]], bufs[0], sems[0])
        for c in range(ch):
            if c + 1 < ch:
                nxt = (c + 1) % 2
                cps[nxt] = pltpu.async_copy(
                    table_hbm.at[idx_v.at[c + 1]], bufs[nxt], sems[nxt])
            cps[c % 2].wait()
            pltpu.sync_copy(bufs[c % 2], out_hbm.at[pl.ds(base + c * cb, cb)])

    return k(idx3, table)


def _tc_body(ef_ref, src_ref, dst_ref, shT_ref, nodeT_ref, l1fT_ref,
             WpT_ref, bp_ref, Wqd0T_ref,
             lngk_ref, lnbk_ref, Wk1T_ref, bk1_ref, Wk2T_ref, bk2_ref,
             lngv_ref, lnbv_ref, Wv1T_ref, bv1_ref, Wv2T_ref, bv2_ref,
             WhoT_ref, WpnT_ref, bpnT_ref,
             Wqd1_ref, Whoo_ref,
             outT_ref, l1outT_ref,
             xq_s, acc_s, y_s, wk_s, wv_s, xs_s, qd_s):
    step = pl.program_id(0)
    nstep = pl.num_programs(0)
    f32 = jnp.float32

    @pl.when(step == 0)
    def _pre():
        x0T = jnp.dot(WpT_ref[...], nodeT_ref[...],
                      preferred_element_type=f32) + bp_ref[...]
        x1T = l1fT_ref[...]
        qd0T = jnp.dot(Wqd0T_ref[...], x0T, preferred_element_type=f32)
        xq_s[0:16, :] = x0T
        xq_s[16:28, :] = x1T
        xq_s[28:32, :] = jnp.zeros((4, 768), f32)
        xq_s[32:48, :] = qd0T
        for v in range(4):
            for i in range(3):
                r = sum(Wqd1_ref[w, v] * x1T[3 * w + i:3 * w + i + 1, :]
                        for w in range(4))
                xq_s[48 + 3 * v + i:49 + 3 * v + i, :] = r
        xq_s[60:64, :] = jnp.zeros((4, 768), f32)
        acc_s[...] = jnp.zeros((48, 768), f32)
        y_s[28:32, :] = jnp.zeros((4, _EB), f32)
        y_s[46:48, :] = jnp.zeros((2, _EB), f32)

    efb = ef_ref[...]
    mu = jnp.mean(efb, axis=1, keepdims=True)
    var = jnp.mean((efb - mu) ** 2, axis=1, keepdims=True)
    nrm = (efb - mu) * lax.rsqrt(var + 1e-5)
    tdims = (((1,), (1,)), ((), ()))
    bf16 = jnp.bfloat16

    lnk = (nrm * lngk_ref[...] + lnbk_ref[...]).astype(bf16)
    hk = jnp.maximum(lax.dot_general(Wk1T_ref[...].astype(bf16), lnk, tdims,
                                     preferred_element_type=f32)
                     + bk1_ref[...], 0.0)
    wk_s[...] = jnp.dot(Wk2T_ref[...].astype(bf16), hk.astype(bf16),
                        preferred_element_type=f32) + bk2_ref[...]

    lnv = (nrm * lngv_ref[...] + lnbv_ref[...]).astype(bf16)
    hv = jnp.maximum(lax.dot_general(Wv1T_ref[...].astype(bf16), lnv, tdims,
                                     preferred_element_type=f32)
                     + bv1_ref[...], 0.0)
    wv_s[...] = jnp.dot(Wv2T_ref[...].astype(bf16), hv.astype(bf16),
                        preferred_element_type=f32) + bv2_ref[...]

    srcb = src_ref[0]
    dstb = dst_ref[0]
    iota = lax.broadcasted_iota(jnp.int32, (768, _EB), 0)
    ohs = (iota == srcb).astype(f32)
    ohd = (iota == dstb).astype(f32)
    xs_s[...] = jnp.dot(xq_s[0:28, :], ohs, preferred_element_type=f32)
    qd_s[...] = jnp.dot(xq_s[32:60, :], ohd, preferred_element_type=f32)
    xsT = xs_s
    qdT = qd_s
    wkT = wk_s
    wvT = wv_s

    shT = shT_ref[...]
    sh0 = shT[0:1]
    sh1 = [shT[1 + i:2 + i] for i in range(3)]
    a_ = shT[4:5]; b_ = shT[5:6]; c_ = shT[6:7]; d_ = shT[7:8]; e_ = shT[8:9]

    x0T = xsT[0:16]
    x1r = [xsT[16 + k:17 + k] for k in range(12)]
    x0sT = x0T * sh0
    inv3 = 1.0 / _S3
    dot11r = [(x1r[3*u] * sh1[0] + x1r[3*u+1] * sh1[1] + x1r[3*u+2] * sh1[2])
              * inv3 for u in range(4)]
    m00 = e_ - c_ * inv3
    m11 = -e_ - c_ * inv3
    m22 = c_ * (2.0 * inv3)
    i5 = 1.0 / (5.0 ** 0.5)
    i2 = 1.0 / (2.0 ** 0.5)
    m12r, y1r, crr = [], [], []
    for u in range(4):
        p0, p1, p2 = x1r[3*u], x1r[3*u+1], x1r[3*u+2]
        m12r += [(m00*p0 + a_*p1 + d_*p2) * i5,
                 (a_*p0 + m11*p1 + b_*p2) * i5,
                 (d_*p0 + b_*p1 + m22*p2) * i5]
        y1r += [p0 * sh0, p1 * sh0, p2 * sh0]
        crr += [(p1*sh1[2] - p2*sh1[1]) * i2,
                (p2*sh1[0] - p0*sh1[2]) * i2,
                (p0*sh1[1] - p1*sh1[0]) * i2]

    i20 = 1.0 / (20.0 ** 0.5)
    i24 = 1.0 / (24.0 ** 0.5)

    def fctp(wT, need_1e):
        out0 = sum(wT[16*u:16*u+16] * x0sT[u:u+1] for u in range(16))
        out0 = out0 + sum(wT[256+16*u:256+16*u+16] * dot11r[u]
                          for u in range(4))
        out0 = out0 * i20
        t011 = [sum(wT[320+4*u+v:321+4*u+v] * x0T[u:u+1] for u in range(16))
                for v in range(4)]
        o1o = []
        for v in range(4):
            for i in range(3):
                r = t011[v] * sh1[i]
                r = r + sum(wT[384+4*u+v:385+4*u+v] * y1r[3*u+i]
                            for u in range(4))
                r = r + sum(wT[400+4*u+v:401+4*u+v] * m12r[3*u+i]
                            for u in range(4))
                o1o.append(r * i24)
        if not need_1e:
            return out0, o1o, None
        o1e = [sum(wT[416+4*u+v:417+4*u+v] * crr[3*u+i] for u in range(4))
               * 0.5
               for v in range(4) for i in range(3)]
        return out0, o1o, o1e

    k0, k1o, _ = fctp(wkT, False)
    v0, v1o, v1e = fctp(wvT, True)

    d0 = jnp.sum(qdT[0:16] * k0, axis=0, keepdims=True)
    d1 = sum(qdT[16+k:17+k] * k1o[k] for k in range(12)) * inv3
    dd = (d0 + d1) * (1.0 / (288.0 ** 0.5))
    ex = jnp.exp(dd * (1.0 / (40.0 ** 0.5)))

    y_s[0:16, :] = v0 * ex
    for k in range(12):
        y_s[16 + k:17 + k, :] = v1o[k] * ex
        y_s[32 + k:33 + k, :] = v1e[k] * ex
    y_s[44:45, :] = ex
    y_s[45:46, :] = jnp.ones((1, _EB), f32)
    acc_s[...] = acc_s[...] + lax.dot_general(
        y_s[...], ohd, (((1,), (1,)), ((), ())), preferred_element_type=f32)

    @pl.when(step == nstep - 1)
    def _post():
        acc = acc_s[...]
        z = acc[44:45]
        cnt = acc[45:46]
        scale = 1.0 / ((z + 1e-5) * jnp.maximum(cnt, 1.0))
        o0T = acc[0:16] * scale
        f0T = jnp.dot(WhoT_ref[...], o0T, preferred_element_type=f32)
        node_outT = (jnp.dot(WpnT_ref[...], f0T, preferred_element_type=f32)
                     + bpnT_ref[...] + nodeT_ref[...])
        outT_ref[...] = node_outT
        for v in range(4):
            for i in range(3):
                r = sum(Whoo_ref[w, v] * acc[16+3*w+i:17+3*w+i]
                        for w in range(4)) * scale
                l1outT_ref[3*v+i:3*v+i+1, :] = r + l1fT_ref[3*v+i:3*v+i+1, :]


def _tc_main(ef, src3, dst3, shT, nodeT, l1fT, consts):
    (WpT, bp_c, Wqd0T, lngk, lnbk, Wk1T, bk1c, Wk2T, bk2c,
     lngv, lnbv, Wv1T, bv1c, Wv2T, bv2c, WhoT, WpnT, bpnT,
     Wqd1, Whoo) = consts
    nblk = ef.shape[0] // _EB
    whole = lambda s: pl.BlockSpec(s, lambda i: (0, 0))
    in_specs = [
        pl.BlockSpec((_EB, 128), lambda i: (i, 0)),
        pl.BlockSpec((1, 1, _EB), lambda i: (i, 0, 0)),
        pl.BlockSpec((1, 1, _EB), lambda i: (i, 0, 0)),
        pl.BlockSpec((9, _EB), lambda i: (0, i)),
        whole((256, 768)), whole((12, 768)),
        whole(WpT.shape), whole(bp_c.shape), whole(Wqd0T.shape),
        whole(lngk.shape), whole(lnbk.shape), whole(Wk1T.shape),
        whole(bk1c.shape), whole(Wk2T.shape), whole(bk2c.shape),
        whole(lngv.shape), whole(lnbv.shape), whole(Wv1T.shape),
        whole(bv1c.shape), whole(Wv2T.shape), whole(bv2c.shape),
        whole(WhoT.shape), whole(WpnT.shape), whole(bpnT.shape),
        pl.BlockSpec(memory_space=pltpu.SMEM),
        pl.BlockSpec(memory_space=pltpu.SMEM),
    ]
    out_specs = [whole((256, 768)), whole((12, 768))]
    return pl.pallas_call(
        _tc_body,
        grid=(nblk,),
        in_specs=in_specs,
        out_specs=out_specs,
        out_shape=[jax.ShapeDtypeStruct((256, 768), jnp.float32),
                   jax.ShapeDtypeStruct((12, 768), jnp.float32)],
        scratch_shapes=[pltpu.VMEM((64, 768), jnp.float32),
                        pltpu.VMEM((48, 768), jnp.float32),
                        pltpu.VMEM((48, _EB), jnp.float32),
                        pltpu.VMEM((432, _EB), jnp.float32),
                        pltpu.VMEM((432, _EB), jnp.float32),
                        pltpu.VMEM((28, _EB), jnp.float32),
                        pltpu.VMEM((28, _EB), jnp.float32)],
        compiler_params=pltpu.CompilerParams(
            dimension_semantics=("arbitrary",)),
    )(ef, src3, dst3, shT, nodeT, l1fT,
      WpT, bp_c, Wqd0T, lngk, lnbk, Wk1T, bk1c, Wk2T, bk2c,
      lngv, lnbv, Wv1T, bv1c, Wv2T, bv2c, WhoT, WpnT, bpnT, Wqd1, Whoo)


def kernel(node, pair, l1_feats, pair_index, edge_src, edge_dst, edge_sh,
           Wp, bp, Wq0, Wq1, ln_g_k, ln_b_k, Wk1, bk1, Wk2, bk2,
           ln_g_v, ln_b_v, Wv1, bv1, Wv2, bv2, Wd0, Wd1o, Wd1e,
           Wh0, Wh1o, Wh1e, Wo0, Wo1o, Wpn, bpn):
    B, L = node.shape[0], node.shape[1]
    n = B * L
    E = edge_src.shape[0]

    flat = (pair_index[0] * (L * L) + pair_index[1] * L
            + pair_index[2]).astype(jnp.int32)
    ch = E // (_NW * _CB)
    ef = _sc_gather_rows(pair.reshape(B * L * L, 128),
                         flat.reshape(_NW, ch, _CB))

    f32 = jnp.float32
    src3 = edge_src.reshape(E // _EB, 1, _EB)
    dst3 = edge_dst.reshape(E // _EB, 1, _EB)
    shT = edge_sh.T
    nodeT = node.reshape(n, 256).T
    l1fT = l1_feats.reshape(n, 12).T
    consts = (
        Wp.T, bp.reshape(16, 1),
        (Wq0 @ Wd0).T / 4.0,
        ln_g_k.reshape(1, 128), ln_b_k.reshape(1, 128),
        Wk1.T, bk1.reshape(128, 1), Wk2.T, bk2.reshape(432, 1),
        ln_g_v.reshape(1, 128), ln_b_v.reshape(1, 128),
        Wv1.T, bv1.reshape(128, 1), Wv2.T, bv2.reshape(432, 1),
        (Wh0 @ Wo0).T / 16.0,
        Wpn.T, bpn.reshape(256, 1),
        (Wq1 @ Wd1o) / 2.0,
        (Wh1o @ Wo1o) / 4.0,
    )
    consts = tuple(c.astype(f32) for c in consts)

    outT, l1outT = _tc_main(ef, src3, dst3, shT, nodeT, l1fT, consts)
    node_out = outT.T.reshape(B, L, 256)
    l1_out = l1outT.T.reshape(B, L, 12)
    return node_out, l1_out
